# baseline (speedup 1.0000x reference)
import functools

import jax
import jax.numpy as jnp
from jax import lax
from jax.experimental import pallas as pl
from jax.experimental.pallas import tpu as pltpu
from jax.experimental.pallas import tpu_sc as plsc

NODES = 50000
DIM = 64
EDGES = 800000

_NC = 2
_NS = 16
_NW = _NC * _NS
_EPW = EDGES // _NW
_C = 384
_NFULL = _EPW // _C
_REM = _EPW - _NFULL * _C

_PACK_BLK = 1024


def _pack_body(wt_ref, out_ref):
    x = lax.bitcast_convert_type(wt_ref[...], jnp.int32)
    rne = x + jnp.int32(0x7FFF) + ((x >> 16) & jnp.int32(1))
    lo = lax.shift_right_logical(rne[:DIM // 2, :], 16)
    hi = rne[DIM // 2:, :] & jnp.int32(-65536)
    out_ref[...] = (lo | hi).T


def _pack_table(weight):
    grid = (NODES + _PACK_BLK - 1) // _PACK_BLK
    return pl.pallas_call(
        _pack_body,
        grid=(grid,),
        in_specs=[pl.BlockSpec((DIM, _PACK_BLK), lambda i: (0, i))],
        out_specs=pl.BlockSpec((_PACK_BLK, DIM // 2), lambda i: (i, 0)),
        out_shape=jax.ShapeDtypeStruct((NODES, DIM // 2), jnp.int32),
    )(weight.T)


@functools.partial(
    pl.kernel,
    out_type=jax.ShapeDtypeStruct((EDGES,), jnp.float32),
    mesh=plsc.VectorSubcoreMesh(core_axis_name="c", subcore_axis_name="s"),
    compiler_params=pltpu.CompilerParams(use_tc_tiling_on_sc=False),
    scratch_types=[
        pltpu.VMEM((_EPW,), jnp.int32),
        pltpu.VMEM((_EPW,), jnp.int32),
        pltpu.VMEM((_C, DIM // 2), jnp.int32),
        pltpu.VMEM((_C, DIM // 2), jnp.int32),
        pltpu.VMEM((_C, DIM // 2), jnp.int32),
        pltpu.VMEM((_C, DIM // 2), jnp.int32),
        pltpu.VMEM((_EPW,), jnp.float32),
        pltpu.SemaphoreType.DMA,
        pltpu.SemaphoreType.DMA,
        pltpu.SemaphoreType.DMA,
        pltpu.SemaphoreType.DMA,
    ],
)
def _edge_dot(eli_hbm, w_hbm, out_hbm,
              idx_s, idx_d, rs0, rd0, rs1, rd1, out_v,
              ss0, sd0, ss1, sd1):
    wid = lax.axis_index("s") * _NC + lax.axis_index("c")
    base0 = wid * _EPW

    pltpu.sync_copy(eli_hbm.at[0, pl.ds(base0, _EPW)], idx_s)
    pltpu.sync_copy(eli_hbm.at[1, pl.ds(base0, _EPW)], idx_d)

    lane = lax.iota(jnp.int32, 16)
    rot_idx = {s: ((lane + s) & 15).reshape(16, 1)
               for s in (1, 2, 4, 8, 12, 14, 15)}
    mask = {s: (lane % (2 * s)) < s for s in (1, 2, 4, 8)}
    _dnums = lax.GatherDimensionNumbers(
        offset_dims=(), collapsed_slice_dims=(0,), start_index_map=(0,))

    def rotv(x, s):
        return lax.gather(x, rot_idx[s], _dnums, (1,),
                          mode=lax.GatherScatterMode.PROMISE_IN_BOUNDS)

    _ORDER = (0, 8, 4, 12, 2, 10, 6, 14, 1, 9, 5, 13, 3, 11, 7, 15)

    def start(lb, n, bs, bd, ss, sd):
        for o in range(0, n, 128):
            m = min(128, n - o)
            pltpu.async_copy(
                w_hbm.at[idx_s.at[pl.ds(lb + o, m)]], bs.at[pl.ds(o, m)], ss)
            pltpu.async_copy(
                w_hbm.at[idx_d.at[pl.ds(lb + o, m)]], bd.at[pl.ds(o, m)], sd)

    def wait(n, bs, bd, ss, sd):
        pltpu.make_async_copy(
            w_hbm.at[idx_s.at[pl.ds(0, n)]], bs.at[pl.ds(0, n)], ss).wait()
        pltpu.make_async_copy(
            w_hbm.at[idx_d.at[pl.ds(0, n)]], bd.at[pl.ds(0, n)], sd).wait()

    def compute(local_base, bs, bd, ngroups, tail):
        def unpack2(w):
            lo = lax.bitcast_convert_type(w << 16, jnp.float32)
            hi = lax.bitcast_convert_type(w & jnp.int32(-65536), jnp.float32)
            return lo, hi

        def edge_partials(e):
            a0l, a0h = unpack2(bs[e, pl.ds(0, 16)])
            b0l, b0h = unpack2(bd[e, pl.ds(0, 16)])
            a1l, a1h = unpack2(bs[e, pl.ds(16, 16)])
            b1l, b1h = unpack2(bd[e, pl.ds(16, 16)])
            p = a0l * b0l + a0h * b0h
            p += a1l * b1l + a1h * b1h
            return p

        def do_group(start_e):
            v = [edge_partials(start_e + o) for o in _ORDER]
            for s in (8, 4, 2, 1):
                v = [jnp.where(mask[s],
                               v[2 * i] + rotv(v[2 * i], s),
                               v[2 * i + 1] + rotv(v[2 * i + 1], 16 - s))
                     for i in range(len(v) // 2)]
            out_v[pl.ds(local_base + start_e, 16)] = v[0]

        @plsc.parallel_loop(0, ngroups * 16, 16)
        def _group_loop(start_e):
            do_group(start_e)
        if tail:
            do_group(ngroups * 16 + tail - 16)

    start(0, _C, rs0, rd0, ss0, sd0)

    def pair_body(k, _):
        c0 = (2 * k) * _C
        start(c0 + _C, _C, rs1, rd1, ss1, sd1)
        wait(_C, rs0, rd0, ss0, sd0)
        compute(c0, rs0, rd0, _C // 16, 0)
        start(c0 + 2 * _C, _C, rs0, rd0, ss0, sd0)
        wait(_C, rs1, rd1, ss1, sd1)
        compute(c0 + _C, rs1, rd1, _C // 16, 0)
        return _

    lax.fori_loop(0, (_NFULL - 1) // 2, pair_body, None)

    last_full = (_NFULL - 1) * _C
    start(_NFULL * _C, _REM, rs1, rd1, ss1, sd1)
    wait(_C, rs0, rd0, ss0, sd0)
    compute(last_full, rs0, rd0, _C // 16, 0)
    wait(_REM, rs1, rd1, ss1, sd1)
    compute(_NFULL * _C, rs1, rd1, _REM // 16, _REM % 16)

    pltpu.sync_copy(out_v, out_hbm.at[pl.ds(base0, _EPW)])


def kernel(edge_label_index, weight):
    wpacked = _pack_table(weight)
    return _edge_dot(edge_label_index, wpacked)


# speedup vs baseline: 2.0873x; 2.0873x over previous
import functools

import jax
import jax.numpy as jnp
from jax import lax
from jax.experimental import pallas as pl
from jax.experimental.pallas import tpu as pltpu
from jax.experimental.pallas import tpu_sc as plsc

NODES = 50000
DIM = 64
EDGES = 800000

_NC = 2
_NS = 16
_NW = _NC * _NS
_EPW = EDGES // _NW
_C = 384
_NFULL = _EPW // _C
_REM = _EPW - _NFULL * _C

_PACK_BLK = 4096


def _pack_body(wt_ref, out_ref):
    x = lax.bitcast_convert_type(wt_ref[...], jnp.int32)
    rne = x + jnp.int32(0x7FFF) + ((x >> 16) & jnp.int32(1))
    lo = lax.shift_right_logical(rne[:DIM // 2, :], 16)
    hi = rne[DIM // 2:, :] & jnp.int32(-65536)
    out_ref[...] = (lo | hi).T


def _pack_table(weight):
    grid = (NODES + _PACK_BLK - 1) // _PACK_BLK
    return pl.pallas_call(
        _pack_body,
        grid=(grid,),
        in_specs=[pl.BlockSpec((DIM, _PACK_BLK), lambda i: (0, i))],
        out_specs=pl.BlockSpec((_PACK_BLK, DIM // 2), lambda i: (i, 0)),
        out_shape=jax.ShapeDtypeStruct((NODES, DIM // 2), jnp.int32),
    )(weight.T)


@functools.partial(
    pl.kernel,
    out_type=jax.ShapeDtypeStruct((EDGES,), jnp.float32),
    mesh=plsc.VectorSubcoreMesh(core_axis_name="c", subcore_axis_name="s"),
    compiler_params=pltpu.CompilerParams(use_tc_tiling_on_sc=False),
    scratch_types=[
        pltpu.VMEM((_EPW,), jnp.int32),
        pltpu.VMEM((_EPW,), jnp.int32),
        pltpu.VMEM((_C, DIM // 2), jnp.int32),
        pltpu.VMEM((_C, DIM // 2), jnp.int32),
        pltpu.VMEM((_C, DIM // 2), jnp.int32),
        pltpu.VMEM((_C, DIM // 2), jnp.int32),
        pltpu.VMEM((_EPW,), jnp.float32),
        pltpu.SemaphoreType.DMA,
        pltpu.SemaphoreType.DMA,
        pltpu.SemaphoreType.DMA,
        pltpu.SemaphoreType.DMA,
    ],
)
def _edge_dot(eli_hbm, w_hbm, out_hbm,
              idx_s, idx_d, rs0, rd0, rs1, rd1, out_v,
              ss0, sd0, ss1, sd1):
    wid = lax.axis_index("s") * _NC + lax.axis_index("c")
    base0 = wid * _EPW

    pltpu.sync_copy(eli_hbm.at[0, pl.ds(base0, _EPW)], idx_s)
    pltpu.sync_copy(eli_hbm.at[1, pl.ds(base0, _EPW)], idx_d)

    lane = lax.iota(jnp.int32, 16)
    rot_idx = {s: ((lane + s) & 15).reshape(16, 1)
               for s in (1, 2, 4, 8, 12, 14, 15)}
    mask = {s: (lane % (2 * s)) < s for s in (1, 2, 4, 8)}
    _dnums = lax.GatherDimensionNumbers(
        offset_dims=(), collapsed_slice_dims=(0,), start_index_map=(0,))

    def rotv(x, s):
        return lax.gather(x, rot_idx[s], _dnums, (1,),
                          mode=lax.GatherScatterMode.PROMISE_IN_BOUNDS)

    _ORDER = (0, 8, 4, 12, 2, 10, 6, 14, 1, 9, 5, 13, 3, 11, 7, 15)

    def start(lb, n, bs, bd, ss, sd):
        for o in range(0, n, 128):
            m = min(128, n - o)
            pltpu.async_copy(
                w_hbm.at[idx_s.at[pl.ds(lb + o, m)]], bs.at[pl.ds(o, m)], ss)
            pltpu.async_copy(
                w_hbm.at[idx_d.at[pl.ds(lb + o, m)]], bd.at[pl.ds(o, m)], sd)

    def wait(n, bs, bd, ss, sd):
        pltpu.make_async_copy(
            w_hbm.at[idx_s.at[pl.ds(0, n)]], bs.at[pl.ds(0, n)], ss).wait()
        pltpu.make_async_copy(
            w_hbm.at[idx_d.at[pl.ds(0, n)]], bd.at[pl.ds(0, n)], sd).wait()

    def compute(local_base, bs, bd, ngroups, tail):
        def unpack2(w):
            lo = lax.bitcast_convert_type(w << 16, jnp.float32)
            hi = lax.bitcast_convert_type(w, jnp.float32)
            return lo, hi

        def edge_partials(e):
            a0l, a0h = unpack2(bs[e, pl.ds(0, 16)])
            b0l, b0h = unpack2(bd[e, pl.ds(0, 16)])
            a1l, a1h = unpack2(bs[e, pl.ds(16, 16)])
            b1l, b1h = unpack2(bd[e, pl.ds(16, 16)])
            p = a0l * b0l + a0h * b0h
            p += a1l * b1l + a1h * b1h
            return p

        def do_group(start_e):
            v = [edge_partials(start_e + o) for o in _ORDER]
            for s in (8, 4, 2, 1):
                v = [jnp.where(mask[s],
                               v[2 * i] + rotv(v[2 * i], s),
                               v[2 * i + 1] + rotv(v[2 * i + 1], 16 - s))
                     for i in range(len(v) // 2)]
            out_v[pl.ds(local_base + start_e, 16)] = v[0]

        lax.fori_loop(0, ngroups, lambda g, _: (do_group(g * 16), _)[1], None)
        if tail:
            do_group(ngroups * 16 + tail - 16)

    start(0, _C, rs0, rd0, ss0, sd0)

    def pair_body(k, _):
        c0 = (2 * k) * _C
        start(c0 + _C, _C, rs1, rd1, ss1, sd1)
        wait(_C, rs0, rd0, ss0, sd0)
        compute(c0, rs0, rd0, _C // 16, 0)
        start(c0 + 2 * _C, _C, rs0, rd0, ss0, sd0)
        wait(_C, rs1, rd1, ss1, sd1)
        compute(c0 + _C, rs1, rd1, _C // 16, 0)
        return _

    lax.fori_loop(0, (_NFULL - 1) // 2, pair_body, None)

    last_full = (_NFULL - 1) * _C
    start(_NFULL * _C, _REM, rs1, rd1, ss1, sd1)
    wait(_C, rs0, rd0, ss0, sd0)
    compute(last_full, rs0, rd0, _C // 16, 0)
    wait(_REM, rs1, rd1, ss1, sd1)
    compute(_NFULL * _C, rs1, rd1, _REM // 16, _REM % 16)

    pltpu.sync_copy(out_v, out_hbm.at[pl.ds(base0, _EPW)])


def kernel(edge_label_index, weight):
    wpacked = _pack_table(weight)
    return _edge_dot(edge_label_index, wpacked)


# speedup vs baseline: 2.4242x; 1.1614x over previous
import functools

import jax
import jax.numpy as jnp
from jax import lax
from jax.experimental import pallas as pl
from jax.experimental.pallas import tpu as pltpu
from jax.experimental.pallas import tpu_sc as plsc

NODES = 50000
DIM = 64
EDGES = 800000

_NC = 2
_NS = 16
_NW = _NC * _NS
_EPW = EDGES // _NW
_C = 384
_NFULL = _EPW // _C
_REM = _EPW - _NFULL * _C

_PROWS = 12544
_PNODES = 4 * _PROWS
_PACK_RB = 1792


def _pack_body(w0_ref, w1_ref, w2_ref, w3_ref, out_ref):
    for k, ref in enumerate((w0_ref, w1_ref, w2_ref, w3_ref)):
        x = lax.bitcast_convert_type(ref[...], jnp.int32)
        rne = x + jnp.int32(0x7FFF) + ((x >> 16) & jnp.int32(1))
        lo = lax.shift_right_logical(rne[:DIM // 2, :], 16)
        hi = rne[DIM // 2:, :] & jnp.int32(-65536)
        out_ref[:, 32 * k:32 * (k + 1)] = (lo | hi).T


def _pack_table(weight):
    def spec(k):
        return pl.BlockSpec((DIM, _PACK_RB), lambda i, kk=k: (0, kk * 7 + i))
    return pl.pallas_call(
        _pack_body,
        grid=(_PROWS // _PACK_RB,),
        in_specs=[spec(0), spec(1), spec(2), spec(3)],
        out_specs=pl.BlockSpec((_PACK_RB, 128), lambda i: (i, 0)),
        out_shape=jax.ShapeDtypeStruct((_PROWS, 128), jnp.int32),
    )(*([weight.T] * 4))


def _remap_body(eli_ref, so_ref, do_ref):
    def remap(n):
        k = ((n >= _PROWS).astype(jnp.int32)
             + (n >= 2 * _PROWS).astype(jnp.int32)
             + (n >= 3 * _PROWS).astype(jnp.int32))
        return ((n - k * _PROWS) << 2) + k

    m = remap(eli_ref[...])
    so_ref[...] = m[0, :]
    do_ref[...] = m[1, :]


def _remap_eli(eli):
    return pl.pallas_call(
        _remap_body,
        out_shape=[jax.ShapeDtypeStruct((EDGES,), jnp.int32),
                   jax.ShapeDtypeStruct((EDGES,), jnp.int32)],
    )(eli)


@functools.partial(
    pl.kernel,
    out_type=jax.ShapeDtypeStruct((EDGES,), jnp.float32),
    mesh=plsc.VectorSubcoreMesh(core_axis_name="c", subcore_axis_name="s"),
    compiler_params=pltpu.CompilerParams(use_tc_tiling_on_sc=False),
    scratch_types=[
        pltpu.VMEM((_EPW,), jnp.int32),
        pltpu.VMEM((_EPW,), jnp.int32),
        pltpu.VMEM((_C, DIM // 2), jnp.int32),
        pltpu.VMEM((_C, DIM // 2), jnp.int32),
        pltpu.VMEM((_C, DIM // 2), jnp.int32),
        pltpu.VMEM((_C, DIM // 2), jnp.int32),
        pltpu.VMEM((_EPW,), jnp.float32),
        pltpu.SemaphoreType.DMA,
        pltpu.SemaphoreType.DMA,
        pltpu.SemaphoreType.DMA,
        pltpu.SemaphoreType.DMA,
    ],
)
def _edge_dot(src_hbm, dst_hbm, w_hbm, out_hbm,
              idx_s, idx_d, rs0, rd0, rs1, rd1, out_v,
              ss0, sd0, ss1, sd1):
    wid = lax.axis_index("s") * _NC + lax.axis_index("c")
    base0 = wid * _EPW

    pltpu.sync_copy(src_hbm.at[pl.ds(base0, _EPW)], idx_s)
    pltpu.sync_copy(dst_hbm.at[pl.ds(base0, _EPW)], idx_d)

    lane = lax.iota(jnp.int32, 16)
    rot_idx = {s: ((lane + s) & 15).reshape(16, 1)
               for s in (1, 2, 4, 8, 12, 14, 15)}
    mask = {s: (lane % (2 * s)) < s for s in (1, 2, 4, 8)}
    _dnums = lax.GatherDimensionNumbers(
        offset_dims=(), collapsed_slice_dims=(0,), start_index_map=(0,))

    def rotv(x, s):
        return lax.gather(x, rot_idx[s], _dnums, (1,),
                          mode=lax.GatherScatterMode.PROMISE_IN_BOUNDS)

    _ORDER = (0, 8, 4, 12, 2, 10, 6, 14, 1, 9, 5, 13, 3, 11, 7, 15)

    def start(lb, n, bs, bd, ss, sd):
        for o in range(0, n, 128):
            m = min(128, n - o)
            pltpu.async_copy(
                w_hbm.at[idx_s.at[pl.ds(lb + o, m)]], bs.at[pl.ds(o, m)], ss)
            pltpu.async_copy(
                w_hbm.at[idx_d.at[pl.ds(lb + o, m)]], bd.at[pl.ds(o, m)], sd)

    def wait(n, bs, bd, ss, sd):
        pltpu.make_async_copy(
            w_hbm.at[idx_s.at[pl.ds(0, n)]], bs.at[pl.ds(0, n)], ss).wait()
        pltpu.make_async_copy(
            w_hbm.at[idx_d.at[pl.ds(0, n)]], bd.at[pl.ds(0, n)], sd).wait()

    def compute(local_base, bs, bd, ngroups, tail):
        def unpack2(w):
            lo = lax.bitcast_convert_type(w << 16, jnp.float32)
            hi = lax.bitcast_convert_type(w, jnp.float32)
            return lo, hi

        def edge_partials(e):
            a0l, a0h = unpack2(bs[e, pl.ds(0, 16)])
            b0l, b0h = unpack2(bd[e, pl.ds(0, 16)])
            a1l, a1h = unpack2(bs[e, pl.ds(16, 16)])
            b1l, b1h = unpack2(bd[e, pl.ds(16, 16)])
            p = a0l * b0l + a0h * b0h
            p += a1l * b1l + a1h * b1h
            return p

        def do_group(start_e):
            v = [edge_partials(start_e + o) for o in _ORDER]
            for s in (8, 4, 2, 1):
                v = [jnp.where(mask[s],
                               v[2 * i] + rotv(v[2 * i], s),
                               v[2 * i + 1] + rotv(v[2 * i + 1], 16 - s))
                     for i in range(len(v) // 2)]
            out_v[pl.ds(local_base + start_e, 16)] = v[0]

        lax.fori_loop(0, ngroups, lambda g, _: (do_group(g * 16), _)[1], None)
        if tail:
            do_group(ngroups * 16 + tail - 16)

    start(0, _C, rs0, rd0, ss0, sd0)

    def pair_body(k, _):
        c0 = (2 * k) * _C
        start(c0 + _C, _C, rs1, rd1, ss1, sd1)
        wait(_C, rs0, rd0, ss0, sd0)
        compute(c0, rs0, rd0, _C // 16, 0)
        start(c0 + 2 * _C, _C, rs0, rd0, ss0, sd0)
        wait(_C, rs1, rd1, ss1, sd1)
        compute(c0 + _C, rs1, rd1, _C // 16, 0)
        return _

    lax.fori_loop(0, (_NFULL - 1) // 2, pair_body, None)

    last_full = (_NFULL - 1) * _C
    start(_NFULL * _C, _REM, rs1, rd1, ss1, sd1)
    wait(_C, rs0, rd0, ss0, sd0)
    compute(last_full, rs0, rd0, _C // 16, 0)
    wait(_REM, rs1, rd1, ss1, sd1)
    compute(_NFULL * _C, rs1, rd1, _REM // 16, _REM % 16)

    pltpu.sync_copy(out_v, out_hbm.at[pl.ds(base0, _EPW)])


def kernel(edge_label_index, weight):
    wpacked = _pack_table(weight).reshape(_PNODES, DIM // 2)
    src, dst = _remap_eli(edge_label_index)
    return _edge_dot(src, dst, wpacked)
